# baseline (speedup 1.0000x reference)
import jax
import jax.numpy as jnp
from jax import lax
from jax.experimental import pallas as pl
from jax.experimental.pallas import tpu as pltpu
from jax.experimental.pallas import tpu_sc as plsc

N_NODES = 10000
D_IN = 128
N_HEADS = 8
FEAT = 64
ACCW = 80
HS = 96

NPAD = 10240
ROW_BLK = 1024
GRID_N = NPAD // ROW_BLK

E_RAW = 320000
E_TOT = E_RAW + N_NODES
N_TILES = 32
CHUNK = 128
NB0 = 62
NB1 = 102
TOTCH = 16 * (NB0 + NB1)
E_PAD = TOTCH * CHUNK
ROWS_PER_TILE = NPAD // 16


def _tc_a_body(x_ref, w1_ref, as_ref, ad_ref, si_ref, hs_ref, asadd_ref):
    h = jnp.dot(x_ref[...], w1_ref[...], preferred_element_type=jnp.float32)
    asads = jnp.dot(h, as_ref[...], preferred_element_type=jnp.float32)
    asadd_ref[...] = jnp.dot(h, ad_ref[...], preferred_element_type=jnp.float32)
    asi = jnp.dot(asads, si_ref[...], preferred_element_type=jnp.float32)
    hs_ref[...] = jnp.concatenate([h, asi], axis=1).astype(jnp.bfloat16)


def _tc_c_body(parts_ref, b1_ref, w2_ref, as_ref, ad_ref, e8_ref, si_ref,
               hs_ref, asadd_ref):
    tot = parts_ref[0] + parts_ref[1]
    num = tot[:, :FEAT]
    den8 = tot[:, FEAT:FEAT + N_HEADS]
    den = jnp.dot(den8, e8_ref[...], preferred_element_type=jnp.float32)
    h1o = jnp.maximum(num / (den + 1e-16) + b1_ref[...], 0.0)
    h2 = jnp.dot(h1o, w2_ref[...], preferred_element_type=jnp.float32)
    asads = jnp.dot(h2, as_ref[...], preferred_element_type=jnp.float32)
    asadd_ref[...] = jnp.dot(h2, ad_ref[...], preferred_element_type=jnp.float32)
    asi = jnp.dot(asads, si_ref[...], preferred_element_type=jnp.float32)
    hs_ref[...] = jnp.concatenate([h2, asi], axis=1).astype(jnp.bfloat16)


def _tc_e_body(parts_ref, b2_ref, out_ref):
    tot = parts_ref[0] + parts_ref[1]
    num = tot[:, :FEAT]
    den = tot[:, FEAT:FEAT + 1]
    out_ref[...] = jnp.maximum(num / (den + 1e-16) + b2_ref[...], 0.0)


def _tc_a(x_pad, w1, a_s, a_d, si):
    return pl.pallas_call(
        _tc_a_body,
        grid=(GRID_N,),
        in_specs=[
            pl.BlockSpec((ROW_BLK, D_IN), lambda i: (i, 0)),
            pl.BlockSpec((D_IN, FEAT), lambda i: (0, 0)),
            pl.BlockSpec((FEAT, 16), lambda i: (0, 0)),
            pl.BlockSpec((FEAT, 16), lambda i: (0, 0)),
            pl.BlockSpec((16, 32), lambda i: (0, 0)),
        ],
        out_specs=[
            pl.BlockSpec((ROW_BLK, HS), lambda i: (i, 0)),
            pl.BlockSpec((ROW_BLK, 16), lambda i: (i, 0)),
        ],
        out_shape=[
            jax.ShapeDtypeStruct((NPAD, HS), jnp.bfloat16),
            jax.ShapeDtypeStruct((NPAD, 16), jnp.float32),
        ],
    )(x_pad, w1, a_s, a_d, si)


def _tc_c(parts, b1r, w2, a_s, a_d, e8, si):
    return pl.pallas_call(
        _tc_c_body,
        grid=(GRID_N,),
        in_specs=[
            pl.BlockSpec((2, ROW_BLK, ACCW), lambda i: (0, i, 0)),
            pl.BlockSpec((1, FEAT), lambda i: (0, 0)),
            pl.BlockSpec((FEAT, FEAT), lambda i: (0, 0)),
            pl.BlockSpec((FEAT, 16), lambda i: (0, 0)),
            pl.BlockSpec((FEAT, 16), lambda i: (0, 0)),
            pl.BlockSpec((N_HEADS, FEAT), lambda i: (0, 0)),
            pl.BlockSpec((16, 32), lambda i: (0, 0)),
        ],
        out_specs=[
            pl.BlockSpec((ROW_BLK, HS), lambda i: (i, 0)),
            pl.BlockSpec((ROW_BLK, 16), lambda i: (i, 0)),
        ],
        out_shape=[
            jax.ShapeDtypeStruct((NPAD, HS), jnp.bfloat16),
            jax.ShapeDtypeStruct((NPAD, 16), jnp.float32),
        ],
    )(parts, b1r, w2, a_s, a_d, e8, si)


def _tc_e(parts, b2r):
    return pl.pallas_call(
        _tc_e_body,
        grid=(GRID_N,),
        in_specs=[
            pl.BlockSpec((2, ROW_BLK, ACCW), lambda i: (0, i, 0)),
            pl.BlockSpec((1, FEAT), lambda i: (0, 0)),
        ],
        out_specs=pl.BlockSpec((ROW_BLK, FEAT), lambda i: (i, 0)),
        out_shape=jax.ShapeDtypeStruct((NPAD, FEAT), jnp.float32),
    )(parts, b2r)


def _shuf(v, idx):
    return jnp.take_along_axis(v, idx, axis=0, mode="promise_in_bounds")


def _sc_edge_body(h_hbm, asadd_hbm, src_hbm, dst_hbm, out_hbm,
                  acc, sall, dall,
                  adg0, adg1, hg0, hg1, msg0, msg1,
                  sad0, sad1, sh0, sh1, ssc0, ssc1):
    c = lax.axis_index("c")
    s = lax.axis_index("s")
    wid = c * 16 + s
    lanes = lax.iota(jnp.int32, 16)
    adg = (adg0, adg1)
    hg = (hg0, hg1)
    msg = (msg0, msg1)
    sad = (sad0, sad1)
    sh = (sh0, sh1)
    ssc = (ssc0, ssc1)

    def _zero_row(k, _):
        for j in range(ACCW // 16):
            msg0[k, pl.ds(16 * j, 16)] = jnp.zeros((16,), jnp.float32)
        return 0
    lax.fori_loop(0, CHUNK, _zero_row, 0)
    for r in range(ROWS_PER_TILE // CHUNK):
        pltpu.sync_copy(msg0, acc.at[pl.ds(s * ROWS_PER_TILE + r * CHUNK, CHUNK)])
    plsc.subcore_barrier()

    def _issue(g, b):
        pltpu.async_copy(asadd_hbm.at[dall.at[g]], adg[b], sad[b])
        pltpu.async_copy(h_hbm.at[sall.at[g]], hg[b], sh[b])

    def _run(base, nbc):
        pltpu.sync_copy(src_hbm.at[pl.ds(base, nbc)], sall.at[pl.ds(0, nbc)])
        pltpu.sync_copy(dst_hbm.at[pl.ds(base, nbc)], dall.at[pl.ds(0, nbc)])
        _issue(0, 0)
        _issue(1, 1)

        def _pair(gp, _):
            for b in range(2):
                g = 2 * gp + b
                pltpu.make_async_copy(asadd_hbm.at[dall.at[g]], adg[b], sad[b]).wait()
                pltpu.make_async_copy(h_hbm.at[sall.at[g]], hg[b], sh[b]).wait()

                @pl.when(g >= 2)
                def _():
                    pltpu.make_async_copy(msg[b], acc.at[dall.at[g]], ssc[b]).wait()

                @plsc.parallel_loop(0, CHUNK, 1, unroll=4)
                def _(k):
                    srow = hg[b][k, pl.ds(FEAT, 32)]
                    vas, _ = plsc.unpack(srow, format=plsc.PackFormat.INTERLEAVED)
                    t = vas + adg[b][k, :]
                    t = jnp.where(t >= 0.0, t, 0.2 * t)
                    e = jnp.exp(t)
                    msg[b][k, pl.ds(FEAT, 16)] = e
                    for half in range(2):
                        hrow = hg[b][k, pl.ds(32 * half, 32)]
                        v0, v1 = plsc.unpack(hrow, format=plsc.PackFormat.INTERLEAVED)
                        ej0 = _shuf(e, (lanes >> 3) + 4 * half)
                        ej1 = _shuf(e, (lanes >> 3) + 4 * half + 2)
                        msg[b][k, pl.ds(32 * half, 16)] = v0 * ej0
                        msg[b][k, pl.ds(32 * half + 16, 16)] = v1 * ej1

                pltpu.async_copy(msg[b], acc.at[dall.at[g]], ssc[b], add=True)

                @pl.when(g + 2 < nbc)
                def _():
                    _issue(g + 2, b)
            return 0

        lax.fori_loop(0, nbc // 2, _pair, 0)
        for b in range(2):
            pltpu.make_async_copy(msg[b], acc.at[dall.at[nbc - 2 + b]], ssc[b]).wait()

    @pl.when(c == 0)
    def _():
        _run(s * NB0, NB0)

    @pl.when(c == 1)
    def _():
        _run(16 * NB0 + s * NB1, NB1)

    plsc.subcore_barrier()
    pltpu.sync_copy(acc.at[pl.ds(s * ROWS_PER_TILE, ROWS_PER_TILE)],
                    out_hbm.at[c, pl.ds(s * ROWS_PER_TILE, ROWS_PER_TILE)])


def _sc_edge(h, asadd, src, dst):
    mesh = plsc.VectorSubcoreMesh(core_axis_name="c", subcore_axis_name="s",
                                  num_cores=2, num_subcores=16)
    dma = pltpu.SemaphoreType.DMA
    return pl.kernel(
        _sc_edge_body,
        out_type=jax.ShapeDtypeStruct((2, NPAD, ACCW), jnp.float32),
        mesh=mesh,
        compiler_params=pltpu.CompilerParams(use_tc_tiling_on_sc=False,
                                             needs_layout_passes=False),
        scratch_types=[
            pltpu.VMEM_SHARED((NPAD, ACCW), jnp.float32),
            pltpu.VMEM((max(NB0, NB1), CHUNK), jnp.int32),
            pltpu.VMEM((max(NB0, NB1), CHUNK), jnp.int32),
            pltpu.VMEM((CHUNK, 16), jnp.float32),
            pltpu.VMEM((CHUNK, 16), jnp.float32),
            pltpu.VMEM((CHUNK, HS), jnp.bfloat16),
            pltpu.VMEM((CHUNK, HS), jnp.bfloat16),
            pltpu.VMEM((CHUNK, ACCW), jnp.float32),
            pltpu.VMEM((CHUNK, ACCW), jnp.float32),
            dma, dma, dma, dma, dma, dma,
        ],
    )(h, asadd, src, dst)


def kernel(x, W1, a_src1, a_dst1, b1, W2, a_src2, a_dst2, b2, edge_index):
    f32 = jnp.float32
    x_pad = jnp.zeros((NPAD, D_IN), f32).at[:N_NODES].set(x)

    loop = jnp.arange(N_NODES, dtype=jnp.int32)
    pad = jnp.full((E_PAD - E_TOT,), N_NODES, dtype=jnp.int32)
    src = jnp.concatenate([edge_index[0], loop, pad]).reshape(TOTCH, CHUNK)
    dst = jnp.concatenate([edge_index[1], loop, pad]).reshape(TOTCH, CHUNK)

    eye8 = jnp.eye(N_HEADS, dtype=f32)
    a1s = (a_src1.reshape(N_HEADS, 8)[:, :, None] * eye8[:, None, :]).reshape(FEAT, N_HEADS)
    a1d = (a_dst1.reshape(N_HEADS, 8)[:, :, None] * eye8[:, None, :]).reshape(FEAT, N_HEADS)
    aS1 = jnp.concatenate([a1s, a1d], axis=1)
    aD1 = jnp.concatenate([a1d, a1d], axis=1)
    aS2 = jnp.concatenate([jnp.tile(a_src2.reshape(FEAT, 1), (1, 8)),
                           jnp.tile(a_dst2.reshape(FEAT, 1), (1, 8))], axis=1)
    aD2 = jnp.tile(a_dst2.reshape(FEAT, 1), (1, 16))
    e8 = jnp.kron(eye8, jnp.ones((1, 8), f32))

    perm = jnp.arange(FEAT).reshape(2, 2, 16).transpose(0, 2, 1).reshape(FEAT)
    w1p = W1[:, perm]
    w2p = W2[:, perm]
    aS1p, aD1p = aS1[perm, :], aD1[perm, :]
    aS2p, aD2p = aS2[perm, :], aD2[perm, :]

    si = jnp.zeros((16, 32), f32).at[jnp.arange(16), 2 * jnp.arange(16)].set(1.0)

    h1, asadd1 = _tc_a(x_pad, w1p, aS1p, aD1p, si)
    parts1 = _sc_edge(h1, asadd1, src, dst)
    h2, asadd2 = _tc_c(parts1, b1.reshape(1, FEAT), w2p, aS2p, aD2p, e8, si)
    parts2 = _sc_edge(h2, asadd2, src, dst)
    out = _tc_e(parts2, b2.reshape(1, FEAT))
    return out[:N_NODES]


# speedup vs baseline: 1.1148x; 1.1148x over previous
import jax
import jax.numpy as jnp
from jax import lax
from jax.experimental import pallas as pl
from jax.experimental.pallas import tpu as pltpu
from jax.experimental.pallas import tpu_sc as plsc

N_NODES = 10000
D_IN = 128
N_HEADS = 8
FEAT = 64
ACCW = 80
HS = 96

NPAD = 10240
ROW_BLK = 1024
GRID_N = NPAD // ROW_BLK

E_RAW = 320000
E_TOT = E_RAW + N_NODES
N_TILES = 32
CHUNK = 128
NB0 = 104
NB1 = 60
TOTCH = 16 * (NB0 + NB1)
E_PAD = TOTCH * CHUNK
ROWS_PER_TILE = NPAD // 16


def _tc_a_body(x_ref, w1_ref, as_ref, ad_ref, si_ref, hs_ref, asadd_ref):
    h = jnp.dot(x_ref[...], w1_ref[...], preferred_element_type=jnp.float32)
    asads = jnp.dot(h, as_ref[...], preferred_element_type=jnp.float32)
    asadd_ref[...] = jnp.dot(h, ad_ref[...], preferred_element_type=jnp.float32)
    asi = jnp.dot(asads, si_ref[...], preferred_element_type=jnp.float32)
    hs_ref[...] = jnp.concatenate([h, asi], axis=1).astype(jnp.bfloat16)


def _tc_c_body(parts_ref, b1_ref, w2_ref, as_ref, ad_ref, e8_ref, si_ref,
               hs_ref, asadd_ref):
    tot = parts_ref[0] + parts_ref[1]
    num = tot[:, :FEAT]
    den8 = tot[:, FEAT:FEAT + N_HEADS]
    den = jnp.dot(den8, e8_ref[...], preferred_element_type=jnp.float32)
    h1o = jnp.maximum(num / (den + 1e-16) + b1_ref[...], 0.0)
    h2 = jnp.dot(h1o, w2_ref[...], preferred_element_type=jnp.float32)
    asads = jnp.dot(h2, as_ref[...], preferred_element_type=jnp.float32)
    asadd_ref[...] = jnp.dot(h2, ad_ref[...], preferred_element_type=jnp.float32)
    asi = jnp.dot(asads, si_ref[...], preferred_element_type=jnp.float32)
    hs_ref[...] = jnp.concatenate([h2, asi], axis=1).astype(jnp.bfloat16)


def _tc_e_body(parts_ref, b2_ref, out_ref):
    tot = parts_ref[0] + parts_ref[1]
    num = tot[:, :FEAT]
    den = tot[:, FEAT:FEAT + 1]
    out_ref[...] = jnp.maximum(num / (den + 1e-16) + b2_ref[...], 0.0)


def _tc_a(x_pad, w1, a_s, a_d, si):
    return pl.pallas_call(
        _tc_a_body,
        grid=(GRID_N,),
        in_specs=[
            pl.BlockSpec((ROW_BLK, D_IN), lambda i: (i, 0)),
            pl.BlockSpec((D_IN, FEAT), lambda i: (0, 0)),
            pl.BlockSpec((FEAT, 16), lambda i: (0, 0)),
            pl.BlockSpec((FEAT, 16), lambda i: (0, 0)),
            pl.BlockSpec((16, 32), lambda i: (0, 0)),
        ],
        out_specs=[
            pl.BlockSpec((ROW_BLK, HS), lambda i: (i, 0)),
            pl.BlockSpec((ROW_BLK, 16), lambda i: (i, 0)),
        ],
        out_shape=[
            jax.ShapeDtypeStruct((NPAD, HS), jnp.bfloat16),
            jax.ShapeDtypeStruct((NPAD, 16), jnp.float32),
        ],
    )(x_pad, w1, a_s, a_d, si)


def _tc_c(parts, b1r, w2, a_s, a_d, e8, si):
    return pl.pallas_call(
        _tc_c_body,
        grid=(GRID_N,),
        in_specs=[
            pl.BlockSpec((2, ROW_BLK, ACCW), lambda i: (0, i, 0)),
            pl.BlockSpec((1, FEAT), lambda i: (0, 0)),
            pl.BlockSpec((FEAT, FEAT), lambda i: (0, 0)),
            pl.BlockSpec((FEAT, 16), lambda i: (0, 0)),
            pl.BlockSpec((FEAT, 16), lambda i: (0, 0)),
            pl.BlockSpec((N_HEADS, FEAT), lambda i: (0, 0)),
            pl.BlockSpec((16, 32), lambda i: (0, 0)),
        ],
        out_specs=[
            pl.BlockSpec((ROW_BLK, HS), lambda i: (i, 0)),
            pl.BlockSpec((ROW_BLK, 16), lambda i: (i, 0)),
        ],
        out_shape=[
            jax.ShapeDtypeStruct((NPAD, HS), jnp.bfloat16),
            jax.ShapeDtypeStruct((NPAD, 16), jnp.float32),
        ],
    )(parts, b1r, w2, a_s, a_d, e8, si)


def _tc_e(parts, b2r):
    return pl.pallas_call(
        _tc_e_body,
        grid=(GRID_N,),
        in_specs=[
            pl.BlockSpec((2, ROW_BLK, ACCW), lambda i: (0, i, 0)),
            pl.BlockSpec((1, FEAT), lambda i: (0, 0)),
        ],
        out_specs=pl.BlockSpec((ROW_BLK, FEAT), lambda i: (i, 0)),
        out_shape=jax.ShapeDtypeStruct((NPAD, FEAT), jnp.float32),
    )(parts, b2r)


def _shuf(v, idx):
    return jnp.take_along_axis(v, idx, axis=0, mode="promise_in_bounds")


def _sc_edge_body(h_hbm, asadd_hbm, src_hbm, dst_hbm, out_hbm,
                  acc, sall, dall,
                  adg0, adg1, hg0, hg1, msg0, msg1,
                  sad0, sad1, sh0, sh1, ssc0, ssc1):
    c = lax.axis_index("c")
    s = lax.axis_index("s")
    wid = c * 16 + s
    lanes = lax.iota(jnp.int32, 16)
    adg = (adg0, adg1)
    hg = (hg0, hg1)
    msg = (msg0, msg1)
    sad = (sad0, sad1)
    sh = (sh0, sh1)
    ssc = (ssc0, ssc1)

    def _zero_row(k, _):
        for j in range(ACCW // 16):
            msg0[k, pl.ds(16 * j, 16)] = jnp.zeros((16,), jnp.float32)
        return 0
    lax.fori_loop(0, CHUNK, _zero_row, 0)
    for r in range(ROWS_PER_TILE // CHUNK):
        pltpu.sync_copy(msg0, acc.at[pl.ds(s * ROWS_PER_TILE + r * CHUNK, CHUNK)])
    plsc.subcore_barrier()

    def _issue(g, b):
        pltpu.async_copy(asadd_hbm.at[dall.at[g]], adg[b], sad[b])
        pltpu.async_copy(h_hbm.at[sall.at[g]], hg[b], sh[b])

    def _run(base, nbc):
        pltpu.sync_copy(src_hbm.at[pl.ds(base, nbc)], sall.at[pl.ds(0, nbc)])
        pltpu.sync_copy(dst_hbm.at[pl.ds(base, nbc)], dall.at[pl.ds(0, nbc)])
        _issue(0, 0)
        _issue(1, 1)

        def _pair(gp, _):
            for b in range(2):
                g = 2 * gp + b
                pltpu.make_async_copy(asadd_hbm.at[dall.at[g]], adg[b], sad[b]).wait()
                pltpu.make_async_copy(h_hbm.at[sall.at[g]], hg[b], sh[b]).wait()

                @pl.when(g >= 2)
                def _():
                    pltpu.make_async_copy(msg[b], acc.at[dall.at[g]], ssc[b]).wait()

                @plsc.parallel_loop(0, CHUNK, 1, unroll=4)
                def _(k):
                    srow = hg[b][k, pl.ds(FEAT, 32)]
                    vas, _ = plsc.unpack(srow, format=plsc.PackFormat.INTERLEAVED)
                    t = vas + adg[b][k, :]
                    t = jnp.where(t >= 0.0, t, 0.2 * t)
                    e = jnp.exp(t)
                    msg[b][k, pl.ds(FEAT, 16)] = e
                    for half in range(2):
                        hrow = hg[b][k, pl.ds(32 * half, 32)]
                        v0, v1 = plsc.unpack(hrow, format=plsc.PackFormat.INTERLEAVED)
                        ej0 = _shuf(e, (lanes >> 3) + 4 * half)
                        ej1 = _shuf(e, (lanes >> 3) + 4 * half + 2)
                        msg[b][k, pl.ds(32 * half, 16)] = v0 * ej0
                        msg[b][k, pl.ds(32 * half + 16, 16)] = v1 * ej1

                pltpu.async_copy(msg[b], acc.at[dall.at[g]], ssc[b], add=True)

                @pl.when(g + 2 < nbc)
                def _():
                    _issue(g + 2, b)
            return 0

        lax.fori_loop(0, nbc // 2, _pair, 0)
        for b in range(2):
            pltpu.make_async_copy(msg[b], acc.at[dall.at[nbc - 2 + b]], ssc[b]).wait()

    @pl.when(c == 0)
    def _():
        _run(s * NB0, NB0)

    @pl.when(c == 1)
    def _():
        _run(16 * NB0 + s * NB1, NB1)

    plsc.subcore_barrier()
    pltpu.sync_copy(acc.at[pl.ds(s * ROWS_PER_TILE, ROWS_PER_TILE)],
                    out_hbm.at[c, pl.ds(s * ROWS_PER_TILE, ROWS_PER_TILE)])


def _sc_edge(h, asadd, src, dst):
    mesh = plsc.VectorSubcoreMesh(core_axis_name="c", subcore_axis_name="s",
                                  num_cores=2, num_subcores=16)
    dma = pltpu.SemaphoreType.DMA
    return pl.kernel(
        _sc_edge_body,
        out_type=jax.ShapeDtypeStruct((2, NPAD, ACCW), jnp.float32),
        mesh=mesh,
        compiler_params=pltpu.CompilerParams(use_tc_tiling_on_sc=False,
                                             needs_layout_passes=False),
        scratch_types=[
            pltpu.VMEM_SHARED((NPAD, ACCW), jnp.float32),
            pltpu.VMEM((max(NB0, NB1), CHUNK), jnp.int32),
            pltpu.VMEM((max(NB0, NB1), CHUNK), jnp.int32),
            pltpu.VMEM((CHUNK, 16), jnp.float32),
            pltpu.VMEM((CHUNK, 16), jnp.float32),
            pltpu.VMEM((CHUNK, HS), jnp.bfloat16),
            pltpu.VMEM((CHUNK, HS), jnp.bfloat16),
            pltpu.VMEM((CHUNK, ACCW), jnp.float32),
            pltpu.VMEM((CHUNK, ACCW), jnp.float32),
            dma, dma, dma, dma, dma, dma,
        ],
    )(h, asadd, src, dst)


def kernel(x, W1, a_src1, a_dst1, b1, W2, a_src2, a_dst2, b2, edge_index):
    f32 = jnp.float32
    x_pad = jnp.zeros((NPAD, D_IN), f32).at[:N_NODES].set(x)

    loop = jnp.arange(N_NODES, dtype=jnp.int32)
    pad = jnp.full((E_PAD - E_TOT,), N_NODES, dtype=jnp.int32)
    src = jnp.concatenate([edge_index[0], loop, pad]).reshape(TOTCH, CHUNK)
    dst = jnp.concatenate([edge_index[1], loop, pad]).reshape(TOTCH, CHUNK)

    eye8 = jnp.eye(N_HEADS, dtype=f32)
    a1s = (a_src1.reshape(N_HEADS, 8)[:, :, None] * eye8[:, None, :]).reshape(FEAT, N_HEADS)
    a1d = (a_dst1.reshape(N_HEADS, 8)[:, :, None] * eye8[:, None, :]).reshape(FEAT, N_HEADS)
    aS1 = jnp.concatenate([a1s, a1d], axis=1)
    aD1 = jnp.concatenate([a1d, a1d], axis=1)
    aS2 = jnp.concatenate([jnp.tile(a_src2.reshape(FEAT, 1), (1, 8)),
                           jnp.tile(a_dst2.reshape(FEAT, 1), (1, 8))], axis=1)
    aD2 = jnp.tile(a_dst2.reshape(FEAT, 1), (1, 16))
    e8 = jnp.kron(eye8, jnp.ones((1, 8), f32))

    perm = jnp.arange(FEAT).reshape(2, 2, 16).transpose(0, 2, 1).reshape(FEAT)
    w1p = W1[:, perm]
    w2p = W2[:, perm]
    aS1p, aD1p = aS1[perm, :], aD1[perm, :]
    aS2p, aD2p = aS2[perm, :], aD2[perm, :]

    si = jnp.zeros((16, 32), f32).at[jnp.arange(16), 2 * jnp.arange(16)].set(1.0)

    h1, asadd1 = _tc_a(x_pad, w1p, aS1p, aD1p, si)
    parts1 = _sc_edge(h1, asadd1, src, dst)
    h2, asadd2 = _tc_c(parts1, b1.reshape(1, FEAT), w2p, aS2p, aD2p, e8, si)
    parts2 = _sc_edge(h2, asadd2, src, dst)
    out = _tc_e(parts2, b2.reshape(1, FEAT))
    return out[:N_NODES]
